# Initial kernel scaffold; baseline (speedup 1.0000x reference)
import functools

import jax
import jax.numpy as jnp
from jax import lax
from jax.experimental import pallas as pl
from jax.experimental.pallas import tpu as pltpu
from jax.experimental.pallas import tpu_sc as plsc

N = 100000
E = 3200000
N_PAD = 100096
E_PAD = 3203072
EROWS = E_PAD // 128
W_ROWS = EROWS // 32
CHUNK = 46
NCHUNKS = W_ROWS // CHUNK
R16 = N_PAD // 16
BLK = 5888

_MESH = plsc.VectorSubcoreMesh(core_axis_name="c", subcore_axis_name="s")


def _make_agg(D):
    def body(hn, srcr, dstr, zeros, out, acc, sidx, didx, rows, stage, sem):
        cid = lax.axis_index("c")
        sid = lax.axis_index("s")
        pltpu.sync_copy(zeros, stage)
        pltpu.sync_copy(stage, acc.at[pl.ds(sid * R16, R16)])
        plsc.subcore_barrier()
        base = (cid * 16 + sid) * W_ROWS

        def chunk(c, carry):
            off = base + c * CHUNK
            pltpu.sync_copy(srcr.at[pl.ds(off, CHUNK)], sidx)
            pltpu.sync_copy(dstr.at[pl.ds(off, CHUNK)], didx)

            def batch(j, carry2):
                pltpu.async_copy(hn.at[sidx.at[j]], rows, sem).wait()
                pltpu.sync_copy(rows, acc.at[didx.at[j]], add=True)
                return carry2

            return lax.fori_loop(0, CHUNK, batch, carry)

        lax.fori_loop(0, NCHUNKS, chunk, 0)
        plsc.subcore_barrier()
        pltpu.sync_copy(acc.at[pl.ds(sid * R16, R16)], stage)
        pltpu.sync_copy(stage, out.at[pl.ds(cid * N_PAD + sid * R16, R16)])

    return pl.kernel(
        body,
        out_type=jax.ShapeDtypeStruct((2 * N_PAD, D), jnp.float32),
        mesh=_MESH,
        scratch_types=[
            pltpu.VMEM_SHARED((N_PAD, D), jnp.float32),
            pltpu.VMEM((CHUNK, 128), jnp.int32),
            pltpu.VMEM((CHUNK, 128), jnp.int32),
            pltpu.VMEM((128, D), jnp.float32),
            pltpu.VMEM((R16, D), jnp.float32),
            pltpu.SemaphoreType.DMA,
        ],
    )


def _make_deg():
    def body(dstr, ones, zeros, out, acc, didx, onesv, stage, sem):
        cid = lax.axis_index("c")
        sid = lax.axis_index("s")
        pltpu.sync_copy(ones, onesv)
        pltpu.sync_copy(zeros, stage)
        pltpu.sync_copy(stage, acc.at[pl.ds(sid * R16, R16)])
        plsc.subcore_barrier()
        base = (cid * 16 + sid) * W_ROWS

        def chunk(c, carry):
            off = base + c * CHUNK
            pltpu.sync_copy(dstr.at[pl.ds(off, CHUNK)], didx)

            def batch(j, carry2):
                pltpu.sync_copy(onesv, acc.at[didx.at[j]], add=True)
                return carry2

            return lax.fori_loop(0, CHUNK, batch, carry)

        lax.fori_loop(0, NCHUNKS, chunk, 0)
        plsc.subcore_barrier()
        pltpu.sync_copy(acc.at[pl.ds(sid * R16, R16)], stage)
        pltpu.sync_copy(stage, out.at[pl.ds(cid * N_PAD + sid * R16, R16)])

    return pl.kernel(
        body,
        out_type=jax.ShapeDtypeStruct((2 * N_PAD, 1), jnp.float32),
        mesh=_MESH,
        scratch_types=[
            pltpu.VMEM_SHARED((N_PAD, 1), jnp.float32),
            pltpu.VMEM((CHUNK, 128), jnp.int32),
            pltpu.VMEM((128, 1), jnp.float32),
            pltpu.VMEM((R16, 1), jnp.float32),
            pltpu.SemaphoreType.DMA,
        ],
    )


def _bspec(shape2):
    return pl.BlockSpec((BLK, shape2), lambda i: (i, 0))


def _cspec(r, c):
    return pl.BlockSpec((r, c), lambda i: (0, 0))


def _stage_a(d0, d1, x_pad):
    def body(d0_ref, d1_ref, x_ref, dis_ref, xn_ref):
        dis = lax.rsqrt(d0_ref[...] + d1_ref[...] + 1.0)
        dis_ref[...] = dis
        xn_ref[...] = x_ref[...] * dis

    return pl.pallas_call(
        body,
        grid=(N_PAD // BLK,),
        in_specs=[_bspec(1), _bspec(1), _bspec(4)],
        out_specs=[_bspec(1), _bspec(4)],
        out_shape=[
            jax.ShapeDtypeStruct((N_PAD, 1), jnp.float32),
            jax.ShapeDtypeStruct((N_PAD, 4), jnp.float32),
        ],
    )(d0, d1, x_pad)


def _stage_b(a0, a1, xn, dis, w1, b1, w2):
    def body(a0_ref, a1_ref, xn_ref, dis_ref, w1_ref, b1_ref, w2_ref, hn2_ref):
        g = (a0_ref[...] + a1_ref[...] + xn_ref[...]) * dis_ref[...]
        h1 = jnp.dot(g, w1_ref[...], preferred_element_type=jnp.float32)
        h1 = jnp.maximum(h1 + b1_ref[...], 0.0)
        h2 = jnp.dot(h1, w2_ref[...], preferred_element_type=jnp.float32)
        hn2_ref[...] = h2 * dis_ref[...]

    return pl.pallas_call(
        body,
        grid=(N_PAD // BLK,),
        in_specs=[_bspec(4), _bspec(4), _bspec(4), _bspec(1),
                  _cspec(4, 16), _cspec(1, 16), _cspec(16, 8)],
        out_specs=_bspec(8),
        out_shape=jax.ShapeDtypeStruct((N_PAD, 8), jnp.float32),
    )(a0, a1, xn, dis, w1, b1, w2)


def _stage_c(a0, a1, hn2, dis, b2):
    def body(a0_ref, a1_ref, hn2_ref, dis_ref, b2_ref, out_ref):
        out_ref[...] = ((a0_ref[...] + a1_ref[...] + hn2_ref[...])
                        * dis_ref[...] + b2_ref[...])

    return pl.pallas_call(
        body,
        grid=(N_PAD // BLK,),
        in_specs=[_bspec(8), _bspec(8), _bspec(8), _bspec(1), _cspec(1, 8)],
        out_specs=_bspec(8),
        out_shape=jax.ShapeDtypeStruct((N_PAD, 8), jnp.float32),
    )(a0, a1, hn2, dis, b2)


_deg_call = _make_deg()
_agg4_call = _make_agg(4)
_agg8_call = _make_agg(8)


def kernel(x, edge_index, W1, b1, W2, b2):
    src = edge_index[0]
    dst = edge_index[1]
    pad = jnp.full((E_PAD - E,), N, jnp.int32)
    srcr = jnp.concatenate([src, pad]).reshape(EROWS, 128)
    dstr = jnp.concatenate([dst, pad]).reshape(EROWS, 128)
    x_pad = jnp.pad(x, ((0, N_PAD - N), (0, 1)))
    w1p = jnp.pad(W1, ((0, 1), (0, 0)))
    w2p = jnp.pad(W2, ((0, 0), (0, 1)))
    b1r = b1.reshape(1, 16)
    b2p = jnp.pad(b2, (0, 1)).reshape(1, 8)
    ones = jnp.ones((128, 1), jnp.float32)
    z1 = jnp.zeros((R16, 1), jnp.float32)
    z4 = jnp.zeros((R16, 4), jnp.float32)
    z8 = jnp.zeros((R16, 8), jnp.float32)

    degp = _deg_call(dstr, ones, z1)
    dis, xn = _stage_a(degp[:N_PAD], degp[N_PAD:], x_pad)
    agg1 = _agg4_call(xn, srcr, dstr, z4)
    hn2 = _stage_b(agg1[:N_PAD], agg1[N_PAD:], xn, dis, w1p, b1r, w2p)
    agg2 = _agg8_call(hn2, srcr, dstr, z8)
    outp = _stage_c(agg2[:N_PAD], agg2[N_PAD:], hn2, dis, b2p)
    return outp[:N, :7]


# speedup vs baseline: 35.6278x; 35.6278x over previous
import functools

import jax
import jax.numpy as jnp
from jax import lax
from jax.experimental import pallas as pl
from jax.experimental.pallas import tpu as pltpu
from jax.experimental.pallas import tpu_sc as plsc

N = 100000
E = 3200000
N_PAD = 100096
E_PAD = 3211264
EROWS = E_PAD // 128
W_ROWS = EROWS // 32
CHUNK = 56
NCHUNKS = W_ROWS // CHUNK
R16 = N_PAD // 16
BLK = 5888

_MESH = plsc.VectorSubcoreMesh(core_axis_name="c", subcore_axis_name="s")


def _make_agg(D):
    def body(hn, srcr, dstr, zeros, out, acc, sidx, didx, rows, stage, sem):
        cid = lax.axis_index("c")
        sid = lax.axis_index("s")
        pltpu.sync_copy(zeros, stage)
        pltpu.sync_copy(stage, acc.at[pl.ds(sid * R16, R16)])
        plsc.subcore_barrier()
        base = (cid * 16 + sid) * W_ROWS

        def chunk(c, carry):
            off = base + c * CHUNK
            pltpu.sync_copy(srcr.at[pl.ds(off, CHUNK)], sidx)
            pltpu.sync_copy(dstr.at[pl.ds(off, CHUNK)], didx)

            def batch(j, carry2):
                pltpu.async_copy(hn.at[sidx.at[j]], rows, sem).wait()
                pltpu.sync_copy(rows, acc.at[didx.at[j]], add=True)
                return carry2

            return lax.fori_loop(0, CHUNK, batch, carry)

        lax.fori_loop(0, NCHUNKS, chunk, 0)
        plsc.subcore_barrier()
        pltpu.sync_copy(acc.at[pl.ds(sid * R16, R16)], stage)
        pltpu.sync_copy(stage, out.at[pl.ds(cid * N_PAD + sid * R16, R16)])

    return pl.kernel(
        body,
        out_type=jax.ShapeDtypeStruct((2 * N_PAD, D), jnp.float32),
        mesh=_MESH,
        compiler_params=pltpu.CompilerParams(use_tc_tiling_on_sc=False),
        scratch_types=[
            pltpu.VMEM_SHARED((N_PAD, D), jnp.float32),
            pltpu.VMEM((CHUNK, 128), jnp.int32),
            pltpu.VMEM((CHUNK, 128), jnp.int32),
            pltpu.VMEM((128, D), jnp.float32),
            pltpu.VMEM((R16, D), jnp.float32),
            pltpu.SemaphoreType.DMA,
        ],
    )


def _make_deg():
    def body(dstr, ones, zeros, out, acc, didx, onesv, stage, sem):
        cid = lax.axis_index("c")
        sid = lax.axis_index("s")
        pltpu.sync_copy(ones, onesv)
        pltpu.sync_copy(zeros, stage)
        pltpu.sync_copy(stage, acc.at[pl.ds(sid * R16, R16)])
        plsc.subcore_barrier()
        base = (cid * 16 + sid) * W_ROWS

        def chunk(c, carry):
            off = base + c * CHUNK
            pltpu.sync_copy(dstr.at[pl.ds(off, CHUNK)], didx)

            def batch(j, carry2):
                pltpu.sync_copy(onesv, acc.at[didx.at[j]], add=True)
                return carry2

            return lax.fori_loop(0, CHUNK, batch, carry)

        lax.fori_loop(0, NCHUNKS, chunk, 0)
        plsc.subcore_barrier()
        pltpu.sync_copy(acc.at[pl.ds(sid * R16, R16)], stage)
        pltpu.sync_copy(stage, out.at[pl.ds(cid * N_PAD + sid * R16, R16)])

    return pl.kernel(
        body,
        out_type=jax.ShapeDtypeStruct((2 * N_PAD, 8), jnp.float32),
        mesh=_MESH,
        compiler_params=pltpu.CompilerParams(use_tc_tiling_on_sc=False),
        scratch_types=[
            pltpu.VMEM_SHARED((N_PAD, 8), jnp.float32),
            pltpu.VMEM((CHUNK, 128), jnp.int32),
            pltpu.VMEM((128, 8), jnp.float32),
            pltpu.VMEM((R16, 8), jnp.float32),
            pltpu.SemaphoreType.DMA,
        ],
    )


def _bspec(shape2):
    return pl.BlockSpec((BLK, shape2), lambda i: (i, 0))


def _cspec(r, c):
    return pl.BlockSpec((r, c), lambda i: (0, 0))


def _stage_a(d0, d1, x_pad):
    def body(d0_ref, d1_ref, x_ref, dis_ref, xn_ref):
        dis = lax.rsqrt(d0_ref[...] + d1_ref[...] + 1.0)
        dis_ref[...] = dis
        xn_ref[...] = x_ref[...] * dis

    return pl.pallas_call(
        body,
        grid=(N_PAD // BLK,),
        in_specs=[_bspec(1), _bspec(1), _bspec(8)],
        out_specs=[_bspec(1), _bspec(8)],
        out_shape=[
            jax.ShapeDtypeStruct((N_PAD, 1), jnp.float32),
            jax.ShapeDtypeStruct((N_PAD, 8), jnp.float32),
        ],
    )(d0, d1, x_pad)


def _stage_b(a0, a1, xn, dis, w1, b1, w2):
    def body(a0_ref, a1_ref, xn_ref, dis_ref, w1_ref, b1_ref, w2_ref, hn2_ref):
        g = (a0_ref[...] + a1_ref[...] + xn_ref[...]) * dis_ref[...]
        h1 = jnp.dot(g, w1_ref[...], preferred_element_type=jnp.float32)
        h1 = jnp.maximum(h1 + b1_ref[...], 0.0)
        h2 = jnp.dot(h1, w2_ref[...], preferred_element_type=jnp.float32)
        hn2_ref[...] = h2 * dis_ref[...]

    return pl.pallas_call(
        body,
        grid=(N_PAD // BLK,),
        in_specs=[_bspec(8), _bspec(8), _bspec(8), _bspec(1),
                  _cspec(8, 16), _cspec(1, 16), _cspec(16, 8)],
        out_specs=_bspec(8),
        out_shape=jax.ShapeDtypeStruct((N_PAD, 8), jnp.float32),
    )(a0, a1, xn, dis, w1, b1, w2)


def _stage_c(a0, a1, hn2, dis, b2):
    def body(a0_ref, a1_ref, hn2_ref, dis_ref, b2_ref, out_ref):
        out_ref[...] = ((a0_ref[...] + a1_ref[...] + hn2_ref[...])
                        * dis_ref[...] + b2_ref[...])

    return pl.pallas_call(
        body,
        grid=(N_PAD // BLK,),
        in_specs=[_bspec(8), _bspec(8), _bspec(8), _bspec(1), _cspec(1, 8)],
        out_specs=_bspec(8),
        out_shape=jax.ShapeDtypeStruct((N_PAD, 8), jnp.float32),
    )(a0, a1, hn2, dis, b2)


_deg_call = _make_deg()
_agg8_call = _make_agg(8)


def kernel(x, edge_index, W1, b1, W2, b2):
    src = edge_index[0]
    dst = edge_index[1]
    pad = jnp.full((E_PAD - E,), N, jnp.int32)
    srcr = jnp.concatenate([src, pad]).reshape(EROWS, 128)
    dstr = jnp.concatenate([dst, pad]).reshape(EROWS, 128)
    x_pad = jnp.pad(x, ((0, N_PAD - N), (0, 5)))
    w1p = jnp.pad(W1, ((0, 5), (0, 0)))
    w2p = jnp.pad(W2, ((0, 0), (0, 1)))
    b1r = b1.reshape(1, 16)
    b2p = jnp.pad(b2, (0, 1)).reshape(1, 8)
    ones = jnp.ones((128, 8), jnp.float32)
    z8 = jnp.zeros((R16, 8), jnp.float32)

    degp = _deg_call(dstr, ones, z8)
    dis, xn = _stage_a(degp[:N_PAD, 0:1], degp[N_PAD:, 0:1], x_pad)
    agg1 = _agg8_call(xn, srcr, dstr, z8)
    hn2 = _stage_b(agg1[:N_PAD], agg1[N_PAD:], xn, dis, w1p, b1r, w2p)
    agg2 = _agg8_call(hn2, srcr, dstr, z8)
    outp = _stage_c(agg2[:N_PAD], agg2[N_PAD:], hn2, dis, b2p)
    return outp[:N, :7]


# speedup vs baseline: 37.8444x; 1.0622x over previous
import functools

import jax
import jax.numpy as jnp
from jax import lax
from jax.experimental import pallas as pl
from jax.experimental.pallas import tpu as pltpu
from jax.experimental.pallas import tpu_sc as plsc

N = 100000
E = 3200000
N_PAD = 100096
E_PAD = 3211264
EROWS = E_PAD // 128
W_ROWS = EROWS // 32
CHUNK = 56
NCHUNKS = W_ROWS // CHUNK
R16 = N_PAD // 16
BLK = 5888

_MESH = plsc.VectorSubcoreMesh(core_axis_name="c", subcore_axis_name="s")


def _make_agg(D):
    def body(hn, srcr, dstr, zeros, out, acc, sidx, didx,
             rows0, rows1, stage, sem0, sem1):
        cid = lax.axis_index("c")
        sid = lax.axis_index("s")
        my = pl.ds(sid * R16, R16)
        pltpu.sync_copy(zeros, stage)
        pltpu.sync_copy(stage, acc.at[my])
        plsc.subcore_barrier()
        base = (cid * 16 + sid) * W_ROWS

        def chunk(c, carry):
            off = base + c * CHUNK
            pltpu.sync_copy(srcr.at[pl.ds(off, CHUNK)], sidx)
            pltpu.sync_copy(dstr.at[pl.ds(off, CHUNK)], didx)
            pltpu.async_copy(hn.at[sidx.at[0]], rows0, sem0)

            def pair(k, carry2):
                j0 = 2 * k
                j1 = 2 * k + 1
                pltpu.make_async_copy(hn.at[sidx.at[j0]], rows0, sem0).wait()
                pltpu.async_copy(hn.at[sidx.at[j1]], rows1, sem1)
                pltpu.sync_copy(rows0, acc.at[didx.at[j0]], add=True)
                pltpu.make_async_copy(hn.at[sidx.at[j1]], rows1, sem1).wait()
                j2 = jnp.minimum(j1 + 1, CHUNK - 1)
                pltpu.async_copy(hn.at[sidx.at[j2]], rows0, sem0)
                pltpu.sync_copy(rows1, acc.at[didx.at[j1]], add=True)
                return carry2

            lax.fori_loop(0, CHUNK // 2, pair, carry)
            pltpu.make_async_copy(hn.at[sidx.at[CHUNK - 1]], rows0, sem0).wait()
            return carry

        lax.fori_loop(0, NCHUNKS, chunk, 0)
        plsc.subcore_barrier()
        pltpu.sync_copy(acc.at[my], stage)
        pltpu.sync_copy(stage, out.at[pl.ds(cid * N_PAD + sid * R16, R16)])

    return pl.kernel(
        body,
        out_type=jax.ShapeDtypeStruct((2 * N_PAD, D), jnp.float32),
        mesh=_MESH,
        compiler_params=pltpu.CompilerParams(use_tc_tiling_on_sc=False),
        scratch_types=[
            pltpu.VMEM_SHARED((N_PAD, D), jnp.float32),
            pltpu.VMEM((CHUNK, 128), jnp.int32),
            pltpu.VMEM((CHUNK, 128), jnp.int32),
            pltpu.VMEM((128, D), jnp.float32),
            pltpu.VMEM((128, D), jnp.float32),
            pltpu.VMEM((R16, D), jnp.float32),
            pltpu.SemaphoreType.DMA,
            pltpu.SemaphoreType.DMA,
        ],
    )


def _make_deg():
    def body(dstr, ones, zeros, out, acc, didx, onesv, stage, sem):
        cid = lax.axis_index("c")
        sid = lax.axis_index("s")
        pltpu.sync_copy(ones, onesv)
        pltpu.sync_copy(zeros, stage)
        pltpu.sync_copy(stage, acc.at[pl.ds(sid * R16, R16)])
        plsc.subcore_barrier()
        base = (cid * 16 + sid) * W_ROWS

        def chunk(c, carry):
            off = base + c * CHUNK
            pltpu.sync_copy(dstr.at[pl.ds(off, CHUNK)], didx)

            def batch(j, carry2):
                pltpu.sync_copy(onesv, acc.at[didx.at[j]], add=True)
                return carry2

            return lax.fori_loop(0, CHUNK, batch, carry)

        lax.fori_loop(0, NCHUNKS, chunk, 0)
        plsc.subcore_barrier()
        pltpu.sync_copy(acc.at[pl.ds(sid * R16, R16)], stage)
        pltpu.sync_copy(stage, out.at[pl.ds(cid * N_PAD + sid * R16, R16)])

    return pl.kernel(
        body,
        out_type=jax.ShapeDtypeStruct((2 * N_PAD, 8), jnp.float32),
        mesh=_MESH,
        compiler_params=pltpu.CompilerParams(use_tc_tiling_on_sc=False),
        scratch_types=[
            pltpu.VMEM_SHARED((N_PAD, 8), jnp.float32),
            pltpu.VMEM((CHUNK, 128), jnp.int32),
            pltpu.VMEM((128, 8), jnp.float32),
            pltpu.VMEM((R16, 8), jnp.float32),
            pltpu.SemaphoreType.DMA,
        ],
    )


def _bspec(shape2):
    return pl.BlockSpec((BLK, shape2), lambda i: (i, 0))


def _cspec(r, c):
    return pl.BlockSpec((r, c), lambda i: (0, 0))


def _stage_a(d0, d1, x_pad):
    def body(d0_ref, d1_ref, x_ref, dis_ref, xn_ref):
        dis = lax.rsqrt(d0_ref[...] + d1_ref[...] + 1.0)
        dis_ref[...] = dis
        xn_ref[...] = x_ref[...] * dis

    return pl.pallas_call(
        body,
        grid=(N_PAD // BLK,),
        in_specs=[_bspec(1), _bspec(1), _bspec(8)],
        out_specs=[_bspec(1), _bspec(8)],
        out_shape=[
            jax.ShapeDtypeStruct((N_PAD, 1), jnp.float32),
            jax.ShapeDtypeStruct((N_PAD, 8), jnp.float32),
        ],
    )(d0, d1, x_pad)


def _stage_b(a0, a1, xn, dis, w1, b1, w2):
    def body(a0_ref, a1_ref, xn_ref, dis_ref, w1_ref, b1_ref, w2_ref, hn2_ref):
        g = (a0_ref[...] + a1_ref[...] + xn_ref[...]) * dis_ref[...]
        h1 = jnp.dot(g, w1_ref[...], preferred_element_type=jnp.float32)
        h1 = jnp.maximum(h1 + b1_ref[...], 0.0)
        h2 = jnp.dot(h1, w2_ref[...], preferred_element_type=jnp.float32)
        hn2_ref[...] = h2 * dis_ref[...]

    return pl.pallas_call(
        body,
        grid=(N_PAD // BLK,),
        in_specs=[_bspec(8), _bspec(8), _bspec(8), _bspec(1),
                  _cspec(8, 16), _cspec(1, 16), _cspec(16, 8)],
        out_specs=_bspec(8),
        out_shape=jax.ShapeDtypeStruct((N_PAD, 8), jnp.float32),
    )(a0, a1, xn, dis, w1, b1, w2)


def _stage_c(a0, a1, hn2, dis, b2):
    def body(a0_ref, a1_ref, hn2_ref, dis_ref, b2_ref, out_ref):
        out_ref[...] = ((a0_ref[...] + a1_ref[...] + hn2_ref[...])
                        * dis_ref[...] + b2_ref[...])

    return pl.pallas_call(
        body,
        grid=(N_PAD // BLK,),
        in_specs=[_bspec(8), _bspec(8), _bspec(8), _bspec(1), _cspec(1, 8)],
        out_specs=_bspec(8),
        out_shape=jax.ShapeDtypeStruct((N_PAD, 8), jnp.float32),
    )(a0, a1, hn2, dis, b2)


_deg_call = _make_deg()
_agg8_call = _make_agg(8)


def kernel(x, edge_index, W1, b1, W2, b2):
    src = edge_index[0]
    dst = edge_index[1]
    pad = jnp.full((E_PAD - E,), N, jnp.int32)
    srcr = jnp.concatenate([src, pad]).reshape(EROWS, 128)
    dstr = jnp.concatenate([dst, pad]).reshape(EROWS, 128)
    x_pad = jnp.pad(x, ((0, N_PAD - N), (0, 5)))
    w1p = jnp.pad(W1, ((0, 5), (0, 0)))
    w2p = jnp.pad(W2, ((0, 0), (0, 1)))
    b1r = b1.reshape(1, 16)
    b2p = jnp.pad(b2, (0, 1)).reshape(1, 8)
    ones = jnp.ones((128, 8), jnp.float32)
    z8 = jnp.zeros((R16, 8), jnp.float32)

    degp = _deg_call(dstr, ones, z8)
    dis, xn = _stage_a(degp[:N_PAD, 0:1], degp[N_PAD:, 0:1], x_pad)
    agg1 = _agg8_call(xn, srcr, dstr, z8)
    hn2 = _stage_b(agg1[:N_PAD], agg1[N_PAD:], xn, dis, w1p, b1r, w2p)
    agg2 = _agg8_call(hn2, srcr, dstr, z8)
    outp = _stage_c(agg2[:N_PAD], agg2[N_PAD:], hn2, dis, b2p)
    return outp[:N, :7]


# speedup vs baseline: 57.3212x; 1.5147x over previous
import functools

import jax
import jax.numpy as jnp
from jax import lax
from jax.experimental import pallas as pl
from jax.experimental.pallas import tpu as pltpu
from jax.experimental.pallas import tpu_sc as plsc

N = 100000
E = 3200000
N_PAD = 100096
E_PAD = 3211264
EROWS = E_PAD // 128
W_ROWS = EROWS // 32
CHUNK = 56
NCHUNKS = W_ROWS // CHUNK
R16 = N_PAD // 16
K = 7
NG = CHUNK // K
BLK = 5888

_MESH = plsc.VectorSubcoreMesh(core_axis_name="c", subcore_axis_name="s")


def _make_agg(D):
    def body(hn, srcr, dstr, zeros, out, acc, sidx, didx,
             bufa, bufb, stage, gsa, gsb, ssa, ssb):
        cid = lax.axis_index("c")
        sid = lax.axis_index("s")
        my = pl.ds(sid * R16, R16)
        pltpu.sync_copy(zeros, stage)
        pltpu.sync_copy(stage, acc.at[my])
        plsc.subcore_barrier()
        base = (cid * 16 + sid) * W_ROWS

        def gath(goff, buf, sem):
            for k in range(K):
                pltpu.async_copy(hn.at[sidx.at[goff + k]],
                                 buf.at[pl.ds(k * 128, 128)], sem)

        def scat(goff, buf, sem):
            for k in range(K):
                pltpu.async_copy(buf.at[pl.ds(k * 128, 128)],
                                 acc.at[didx.at[goff + k]], sem, add=True)

        def drain(buf, sem):
            for k in range(K):
                pltpu.make_async_copy(hn.at[pl.ds(0, 128)],
                                      buf.at[pl.ds(k * 128, 128)], sem).wait()

        def chunk(c, carry):
            @pl.when(c > 0)
            def _():
                drain(bufb, ssb)

            off = base + c * CHUNK
            pltpu.sync_copy(srcr.at[pl.ds(off, CHUNK)], sidx)
            pltpu.sync_copy(dstr.at[pl.ds(off, CHUNK)], didx)
            gath(0, bufa, gsa)

            def pair(k, carry2):
                g1 = 2 * k + 1
                drain(bufa, gsa)
                @pl.when(k > 0)
                def _():
                    drain(bufb, ssb)
                gath(g1 * K, bufb, gsb)
                scat(2 * k * K, bufa, ssa)
                drain(bufb, gsb)
                drain(bufa, ssa)
                @pl.when(g1 + 1 < NG)
                def _():
                    gath((g1 + 1) * K, bufa, gsa)
                scat(g1 * K, bufb, ssb)
                return carry2

            return lax.fori_loop(0, NG // 2, pair, carry)

        lax.fori_loop(0, NCHUNKS, chunk, 0)
        drain(bufb, ssb)
        plsc.subcore_barrier()
        pltpu.sync_copy(acc.at[my], stage)
        pltpu.sync_copy(stage, out.at[pl.ds(cid * N_PAD + sid * R16, R16)])

    return pl.kernel(
        body,
        out_type=jax.ShapeDtypeStruct((2 * N_PAD, D), jnp.float32),
        mesh=_MESH,
        compiler_params=pltpu.CompilerParams(use_tc_tiling_on_sc=False),
        scratch_types=[
            pltpu.VMEM_SHARED((N_PAD, D), jnp.float32),
            pltpu.VMEM((CHUNK, 128), jnp.int32),
            pltpu.VMEM((CHUNK, 128), jnp.int32),
            pltpu.VMEM((K * 128, D), jnp.float32),
            pltpu.VMEM((K * 128, D), jnp.float32),
            pltpu.VMEM((R16, D), jnp.float32),
            pltpu.SemaphoreType.DMA,
            pltpu.SemaphoreType.DMA,
            pltpu.SemaphoreType.DMA,
            pltpu.SemaphoreType.DMA,
        ],
    )


def _make_deg():
    def body(dstr, ones, zeros, out, acc, didx, onesv, stage, sem):
        cid = lax.axis_index("c")
        sid = lax.axis_index("s")
        my = pl.ds(sid * R16, R16)
        pltpu.sync_copy(ones, onesv)
        pltpu.sync_copy(zeros, stage)
        pltpu.sync_copy(stage, acc.at[my])
        plsc.subcore_barrier()
        base = (cid * 16 + sid) * W_ROWS

        def drain_all(carry):
            def one(j, carry2):
                pltpu.make_async_copy(ones, onesv, sem).wait()
                return carry2
            return lax.fori_loop(0, CHUNK, one, carry)

        def chunk(c, carry):
            @pl.when(c > 0)
            def _():
                drain_all(0)

            off = base + c * CHUNK
            pltpu.sync_copy(dstr.at[pl.ds(off, CHUNK)], didx)

            def batch(j, carry2):
                pltpu.async_copy(onesv, acc.at[didx.at[j]], sem, add=True)
                return carry2

            return lax.fori_loop(0, CHUNK, batch, carry)

        lax.fori_loop(0, NCHUNKS, chunk, 0)
        drain_all(0)
        plsc.subcore_barrier()
        pltpu.sync_copy(acc.at[my], stage)
        pltpu.sync_copy(stage, out.at[pl.ds(cid * N_PAD + sid * R16, R16)])

    return pl.kernel(
        body,
        out_type=jax.ShapeDtypeStruct((2 * N_PAD, 8), jnp.float32),
        mesh=_MESH,
        compiler_params=pltpu.CompilerParams(use_tc_tiling_on_sc=False),
        scratch_types=[
            pltpu.VMEM_SHARED((N_PAD, 8), jnp.float32),
            pltpu.VMEM((CHUNK, 128), jnp.int32),
            pltpu.VMEM((128, 8), jnp.float32),
            pltpu.VMEM((R16, 8), jnp.float32),
            pltpu.SemaphoreType.DMA,
        ],
    )


def _bspec(shape2):
    return pl.BlockSpec((BLK, shape2), lambda i: (i, 0))


def _cspec(r, c):
    return pl.BlockSpec((r, c), lambda i: (0, 0))


def _stage_a(d0, d1, x_pad):
    def body(d0_ref, d1_ref, x_ref, dis_ref, xn_ref):
        dis = lax.rsqrt(d0_ref[...] + d1_ref[...] + 1.0)
        dis_ref[...] = dis
        xn_ref[...] = x_ref[...] * dis

    return pl.pallas_call(
        body,
        grid=(N_PAD // BLK,),
        in_specs=[_bspec(1), _bspec(1), _bspec(8)],
        out_specs=[_bspec(1), _bspec(8)],
        out_shape=[
            jax.ShapeDtypeStruct((N_PAD, 1), jnp.float32),
            jax.ShapeDtypeStruct((N_PAD, 8), jnp.float32),
        ],
    )(d0, d1, x_pad)


def _stage_b(a0, a1, xn, dis, w1, b1, w2):
    def body(a0_ref, a1_ref, xn_ref, dis_ref, w1_ref, b1_ref, w2_ref, hn2_ref):
        g = (a0_ref[...] + a1_ref[...] + xn_ref[...]) * dis_ref[...]
        h1 = jnp.dot(g, w1_ref[...], preferred_element_type=jnp.float32)
        h1 = jnp.maximum(h1 + b1_ref[...], 0.0)
        h2 = jnp.dot(h1, w2_ref[...], preferred_element_type=jnp.float32)
        hn2_ref[...] = h2 * dis_ref[...]

    return pl.pallas_call(
        body,
        grid=(N_PAD // BLK,),
        in_specs=[_bspec(8), _bspec(8), _bspec(8), _bspec(1),
                  _cspec(8, 16), _cspec(1, 16), _cspec(16, 8)],
        out_specs=_bspec(8),
        out_shape=jax.ShapeDtypeStruct((N_PAD, 8), jnp.float32),
    )(a0, a1, xn, dis, w1, b1, w2)


def _stage_c(a0, a1, hn2, dis, b2):
    def body(a0_ref, a1_ref, hn2_ref, dis_ref, b2_ref, out_ref):
        out_ref[...] = ((a0_ref[...] + a1_ref[...] + hn2_ref[...])
                        * dis_ref[...] + b2_ref[...])

    return pl.pallas_call(
        body,
        grid=(N_PAD // BLK,),
        in_specs=[_bspec(8), _bspec(8), _bspec(8), _bspec(1), _cspec(1, 8)],
        out_specs=_bspec(8),
        out_shape=jax.ShapeDtypeStruct((N_PAD, 8), jnp.float32),
    )(a0, a1, hn2, dis, b2)


_deg_call = _make_deg()
_agg8_call = _make_agg(8)


def kernel(x, edge_index, W1, b1, W2, b2):
    src = edge_index[0]
    dst = edge_index[1]
    pad = jnp.full((E_PAD - E,), N, jnp.int32)
    srcr = jnp.concatenate([src, pad]).reshape(EROWS, 128)
    dstr = jnp.concatenate([dst, pad]).reshape(EROWS, 128)
    x_pad = jnp.pad(x, ((0, N_PAD - N), (0, 5)))
    w1p = jnp.pad(W1, ((0, 5), (0, 0)))
    w2p = jnp.pad(W2, ((0, 0), (0, 1)))
    b1r = b1.reshape(1, 16)
    b2p = jnp.pad(b2, (0, 1)).reshape(1, 8)
    ones = jnp.ones((128, 8), jnp.float32)
    z8 = jnp.zeros((R16, 8), jnp.float32)

    degp = _deg_call(dstr, ones, z8)
    dis, xn = _stage_a(degp[:N_PAD, 0:1], degp[N_PAD:, 0:1], x_pad)
    agg1 = _agg8_call(xn, srcr, dstr, z8)
    hn2 = _stage_b(agg1[:N_PAD], agg1[N_PAD:], xn, dis, w1p, b1r, w2p)
    agg2 = _agg8_call(hn2, srcr, dstr, z8)
    outp = _stage_c(agg2[:N_PAD], agg2[N_PAD:], hn2, dis, b2p)
    return outp[:N, :7]


# speedup vs baseline: 100.2586x; 1.7491x over previous
import functools

import jax
import jax.numpy as jnp
from jax import lax
from jax.experimental import pallas as pl
from jax.experimental.pallas import tpu as pltpu
from jax.experimental.pallas import tpu_sc as plsc

N = 100000
E = 3200000
N_PAD = 100096
E_PAD = 3211264
EROWS = E_PAD // 128
W_ROWS = EROWS // 32
CHUNK = 56
NCHUNKS = W_ROWS // CHUNK
R16 = N_PAD // 16
K = 7
NG = CHUNK // K
BLK = 5888

_MESH = plsc.VectorSubcoreMesh(core_axis_name="c", subcore_axis_name="s")


def _make_agg(D):
    def body(hn, srcr, dstr, zeros, out, acc, sidx, didx,
             bufa, bufb, stage, gsa, gsb, ssa, ssb):
        cid = lax.axis_index("c")
        sid = lax.axis_index("s")
        my = pl.ds(sid * R16, R16)
        pltpu.sync_copy(zeros, stage)
        pltpu.sync_copy(stage, acc.at[my])
        plsc.subcore_barrier()
        base = (cid * 16 + sid) * W_ROWS

        def gath(goff, buf, sem):
            for k in range(K):
                pltpu.async_copy(hn.at[sidx.at[goff + k]],
                                 buf.at[pl.ds(k * 128, 128)], sem)

        def scat(goff, buf, sem):
            for k in range(K):
                pltpu.async_copy(buf.at[pl.ds(k * 128, 128)],
                                 acc.at[didx.at[goff + k]], sem, add=True)

        def drain(buf, sem):
            for k in range(K):
                pltpu.make_async_copy(hn.at[pl.ds(0, 128)],
                                      buf.at[pl.ds(k * 128, 128)], sem).wait()

        def chunk(c, carry):
            @pl.when(c > 0)
            def _():
                drain(bufb, ssb)

            off = base + c * CHUNK
            pltpu.sync_copy(srcr.at[pl.ds(off, CHUNK)], sidx)
            pltpu.sync_copy(dstr.at[pl.ds(off, CHUNK)], didx)
            gath(0, bufa, gsa)

            def pair(k, carry2):
                g1 = 2 * k + 1
                drain(bufa, gsa)
                @pl.when(k > 0)
                def _():
                    drain(bufb, ssb)
                gath(g1 * K, bufb, gsb)
                scat(2 * k * K, bufa, ssa)
                drain(bufb, gsb)
                drain(bufa, ssa)
                @pl.when(g1 + 1 < NG)
                def _():
                    gath((g1 + 1) * K, bufa, gsa)
                scat(g1 * K, bufb, ssb)
                return carry2

            return lax.fori_loop(0, NG // 2, pair, carry)

        lax.fori_loop(0, NCHUNKS, chunk, 0)
        drain(bufb, ssb)
        plsc.subcore_barrier()
        pltpu.sync_copy(acc.at[my], stage)
        pltpu.sync_copy(stage, out.at[pl.ds(cid * N_PAD + sid * R16, R16)])

    return pl.kernel(
        body,
        out_type=jax.ShapeDtypeStruct((2 * N_PAD, D), jnp.float32),
        mesh=_MESH,
        compiler_params=pltpu.CompilerParams(use_tc_tiling_on_sc=False),
        scratch_types=[
            pltpu.VMEM_SHARED((N_PAD, D), jnp.float32),
            pltpu.VMEM((CHUNK, 128), jnp.int32),
            pltpu.VMEM((CHUNK, 128), jnp.int32),
            pltpu.VMEM((K * 128, D), jnp.float32),
            pltpu.VMEM((K * 128, D), jnp.float32),
            pltpu.VMEM((R16, D), jnp.float32),
            pltpu.SemaphoreType.DMA,
            pltpu.SemaphoreType.DMA,
            pltpu.SemaphoreType.DMA,
            pltpu.SemaphoreType.DMA,
        ],
    )


def _make_deg():
    def body(dstr, ones, zeros, out, acc, didx, onesv, stage, sem):
        cid = lax.axis_index("c")
        sid = lax.axis_index("s")
        my = pl.ds(sid * R16, R16)
        pltpu.sync_copy(ones, onesv)
        pltpu.sync_copy(zeros, stage)
        pltpu.sync_copy(stage, acc.at[my])
        plsc.subcore_barrier()
        base = (cid * 16 + sid) * W_ROWS

        def drain_all(carry):
            def one(j, carry2):
                pltpu.make_async_copy(ones, onesv, sem).wait()
                return carry2
            return lax.fori_loop(0, CHUNK, one, carry)

        def chunk(c, carry):
            @pl.when(c > 0)
            def _():
                drain_all(0)

            off = base + c * CHUNK
            pltpu.sync_copy(dstr.at[pl.ds(off, CHUNK)], didx)

            def batch(j, carry2):
                pltpu.async_copy(onesv, acc.at[didx.at[j]], sem, add=True)
                return carry2

            return lax.fori_loop(0, CHUNK, batch, carry)

        lax.fori_loop(0, NCHUNKS, chunk, 0)
        drain_all(0)
        plsc.subcore_barrier()
        pltpu.sync_copy(acc.at[my], stage)
        pltpu.sync_copy(stage, out.at[pl.ds(cid * N_PAD + sid * R16, R16)])

    return pl.kernel(
        body,
        out_type=jax.ShapeDtypeStruct((2 * N_PAD, 8), jnp.float32),
        mesh=_MESH,
        compiler_params=pltpu.CompilerParams(use_tc_tiling_on_sc=False),
        scratch_types=[
            pltpu.VMEM_SHARED((N_PAD, 8), jnp.float32),
            pltpu.VMEM((CHUNK, 128), jnp.int32),
            pltpu.VMEM((128, 8), jnp.float32),
            pltpu.VMEM((R16, 8), jnp.float32),
            pltpu.SemaphoreType.DMA,
        ],
    )


NF = N_PAD * 8 // 128


def _stage_a(degf, xf):
    def body(degf_ref, xf_ref, dis_ref, xn_ref):
        d = degf_ref[0:NF, :] + degf_ref[NF:2 * NF, :] + 1.0
        dis = lax.rsqrt(d)
        dis_ref[...] = dis
        xn_ref[...] = xf_ref[...] * dis

    return pl.pallas_call(
        body,
        out_shape=[
            jax.ShapeDtypeStruct((NF, 128), jnp.float32),
            jax.ShapeDtypeStruct((NF, 128), jnp.float32),
        ],
    )(degf, xf)


def _stage_b(aggf, xnf, disf, bd1, b1f, bd2):
    def body(aggf_ref, xnf_ref, disf_ref, bd1_ref, b1f_ref, bd2_ref, hn2_ref):
        dis = disf_ref[...]
        g = (aggf_ref[0:NF, :] + aggf_ref[NF:2 * NF, :] + xnf_ref[...]) * dis
        h1 = jnp.dot(g, bd1_ref[...], preferred_element_type=jnp.float32)
        h1 = jnp.maximum(h1 + b1f_ref[...], 0.0)
        h2 = jnp.dot(h1, bd2_ref[...], preferred_element_type=jnp.float32)
        hn2_ref[...] = h2 * dis

    return pl.pallas_call(
        body,
        out_shape=jax.ShapeDtypeStruct((NF, 128), jnp.float32),
    )(aggf, xnf, disf, bd1, b1f, bd2)


def _stage_c(aggf, hn2f, disf, b2f):
    def body(aggf_ref, hn2f_ref, disf_ref, b2f_ref, out_ref):
        out_ref[...] = ((aggf_ref[0:NF, :] + aggf_ref[NF:2 * NF, :]
                         + hn2f_ref[...]) * disf_ref[...] + b2f_ref[...])

    return pl.pallas_call(
        body,
        out_shape=jax.ShapeDtypeStruct((NF, 128), jnp.float32),
    )(aggf, hn2f, disf, b2f)


_deg_call = _make_deg()
_agg8_call = _make_agg(8)


def kernel(x, edge_index, W1, b1, W2, b2):
    src = edge_index[0]
    dst = edge_index[1]
    pad = jnp.full((E_PAD - E,), N, jnp.int32)
    srcr = jnp.concatenate([src, pad]).reshape(EROWS, 128)
    dstr = jnp.concatenate([dst, pad]).reshape(EROWS, 128)
    xf = jnp.pad(x, ((0, N_PAD - N), (0, 5))).reshape(NF, 128)
    w1p = jnp.pad(W1, ((0, 5), (0, 0)))
    w2p = jnp.pad(W2, ((0, 0), (0, 1)))
    eye16 = jnp.eye(16, dtype=jnp.float32)
    bd1 = jnp.kron(eye16, w1p)
    bd2 = jnp.kron(eye16, w2p)
    b1f = jnp.tile(b1, 16).reshape(1, 256)
    b2f = jnp.tile(jnp.pad(b2, (0, 1)), 16).reshape(1, 128)
    ones = jnp.ones((128, 8), jnp.float32)
    z8 = jnp.zeros((R16, 8), jnp.float32)

    degp = _deg_call(dstr, ones, z8)
    disf, xnf = _stage_a(degp.reshape(2 * NF, 128), xf)
    agg1 = _agg8_call(xnf.reshape(N_PAD, 8), srcr, dstr, z8)
    hn2f = _stage_b(agg1.reshape(2 * NF, 128), xnf, disf, bd1, b1f, bd2)
    agg2 = _agg8_call(hn2f.reshape(N_PAD, 8), srcr, dstr, z8)
    outf = _stage_c(agg2.reshape(2 * NF, 128), hn2f, disf, b2f)
    return outf.reshape(N_PAD, 8)[:N, :7]


# speedup vs baseline: 102.6063x; 1.0234x over previous
import jax
import jax.numpy as jnp
from jax import lax
from jax.experimental import pallas as pl
from jax.experimental.pallas import tpu as pltpu
from jax.experimental.pallas import tpu_sc as plsc

N = 100000
E = 3200000
N_PAD = 100096
E_PAD = 3211264
EROWS = E_PAD // 128
W_ROWS0 = 840
W_ROWS1 = 728
SPLIT = 16 * W_ROWS0
CHUNK = 56
NCHUNKS0 = W_ROWS0 // CHUNK
NCHUNKS1 = W_ROWS1 // CHUNK
R16 = N_PAD // 16
K = 7
NG = CHUNK // K

_MESH = plsc.VectorSubcoreMesh(core_axis_name="c", subcore_axis_name="s")


def _make_agg(D):
    def body(hn, srcr, dstr, zeros, out0, out1, acc, sidx, didx,
             bufa, bufb, stage, gsa, gsb, ssa, ssb):
        cid = lax.axis_index("c")
        sid = lax.axis_index("s")
        my = pl.ds(sid * R16, R16)
        pltpu.sync_copy(zeros, stage)
        pltpu.sync_copy(stage, acc.at[my])
        plsc.subcore_barrier()
        w_rows = jnp.where(cid == 0, W_ROWS0, W_ROWS1)
        nchunks = jnp.where(cid == 0, NCHUNKS0, NCHUNKS1)
        base = cid * SPLIT + sid * w_rows

        def gath(goff, buf, sem):
            for k in range(K):
                pltpu.async_copy(hn.at[sidx.at[goff + k]],
                                 buf.at[pl.ds(k * 128, 128)], sem)

        def scat(goff, buf, sem):
            for k in range(K):
                pltpu.async_copy(buf.at[pl.ds(k * 128, 128)],
                                 acc.at[didx.at[goff + k]], sem, add=True)

        def drain(buf, sem):
            for k in range(K):
                pltpu.make_async_copy(hn.at[pl.ds(0, 128)],
                                      buf.at[pl.ds(k * 128, 128)], sem).wait()

        def chunk(c, carry):
            @pl.when(c > 0)
            def _():
                drain(bufb, ssb)

            off = base + c * CHUNK
            pltpu.sync_copy(srcr.at[pl.ds(off, CHUNK)], sidx)
            pltpu.sync_copy(dstr.at[pl.ds(off, CHUNK)], didx)
            gath(0, bufa, gsa)

            def pair(k, carry2):
                g1 = 2 * k + 1
                drain(bufa, gsa)
                @pl.when(k > 0)
                def _():
                    drain(bufb, ssb)
                gath(g1 * K, bufb, gsb)
                scat(2 * k * K, bufa, ssa)
                drain(bufb, gsb)
                drain(bufa, ssa)
                @pl.when(g1 + 1 < NG)
                def _():
                    gath((g1 + 1) * K, bufa, gsa)
                scat(g1 * K, bufb, ssb)
                return carry2

            return lax.fori_loop(0, NG // 2, pair, carry)

        lax.fori_loop(0, nchunks, chunk, 0)
        drain(bufb, ssb)
        plsc.subcore_barrier()
        pltpu.sync_copy(acc.at[my], stage)

        @pl.when(cid == 0)
        def _():
            pltpu.sync_copy(stage, out0.at[my])

        @pl.when(cid == 1)
        def _():
            pltpu.sync_copy(stage, out1.at[my])

    return pl.kernel(
        body,
        out_type=(jax.ShapeDtypeStruct((N_PAD, D), jnp.float32),
                  jax.ShapeDtypeStruct((N_PAD, D), jnp.float32)),
        mesh=_MESH,
        compiler_params=pltpu.CompilerParams(use_tc_tiling_on_sc=False),
        scratch_types=[
            pltpu.VMEM_SHARED((N_PAD, D), jnp.float32),
            pltpu.VMEM((CHUNK, 128), jnp.int32),
            pltpu.VMEM((CHUNK, 128), jnp.int32),
            pltpu.VMEM((K * 128, D), jnp.float32),
            pltpu.VMEM((K * 128, D), jnp.float32),
            pltpu.VMEM((R16, D), jnp.float32),
            pltpu.SemaphoreType.DMA,
            pltpu.SemaphoreType.DMA,
            pltpu.SemaphoreType.DMA,
            pltpu.SemaphoreType.DMA,
        ],
    )


def _make_deg():
    def body(dstr, ones, zeros, out0, out1, acc, didx, onesv, stage, sem):
        cid = lax.axis_index("c")
        sid = lax.axis_index("s")
        my = pl.ds(sid * R16, R16)
        pltpu.sync_copy(ones, onesv)
        pltpu.sync_copy(zeros, stage)
        pltpu.sync_copy(stage, acc.at[my])
        plsc.subcore_barrier()
        w_rows = jnp.where(cid == 0, W_ROWS0, W_ROWS1)
        nchunks = jnp.where(cid == 0, NCHUNKS0, NCHUNKS1)
        base = cid * SPLIT + sid * w_rows

        def drain_all(carry):
            def one(j, carry2):
                pltpu.make_async_copy(ones, onesv, sem).wait()
                return carry2
            return lax.fori_loop(0, CHUNK, one, carry)

        def chunk(c, carry):
            @pl.when(c > 0)
            def _():
                drain_all(0)

            off = base + c * CHUNK
            pltpu.sync_copy(dstr.at[pl.ds(off, CHUNK)], didx)

            def batch(j, carry2):
                pltpu.async_copy(onesv, acc.at[didx.at[j]], sem, add=True)
                return carry2

            return lax.fori_loop(0, CHUNK, batch, carry)

        lax.fori_loop(0, nchunks, chunk, 0)
        drain_all(0)
        plsc.subcore_barrier()
        pltpu.sync_copy(acc.at[my], stage)

        @pl.when(cid == 0)
        def _():
            pltpu.sync_copy(stage, out0.at[my])

        @pl.when(cid == 1)
        def _():
            pltpu.sync_copy(stage, out1.at[my])

    return pl.kernel(
        body,
        out_type=(jax.ShapeDtypeStruct((N_PAD, 8), jnp.float32),
                  jax.ShapeDtypeStruct((N_PAD, 8), jnp.float32)),
        mesh=_MESH,
        compiler_params=pltpu.CompilerParams(use_tc_tiling_on_sc=False),
        scratch_types=[
            pltpu.VMEM_SHARED((N_PAD, 8), jnp.float32),
            pltpu.VMEM((CHUNK, 128), jnp.int32),
            pltpu.VMEM((128, 8), jnp.float32),
            pltpu.VMEM((R16, 8), jnp.float32),
            pltpu.SemaphoreType.DMA,
        ],
    )


NF = N_PAD * 8 // 128
CG = 23
CBF = NF // CG
CBN = 16 * CBF


def _stage_a(d0f, d1f, xf):
    def body(d0_ref, d1_ref, xf_ref, dis_ref, xn_ref):
        dis = lax.rsqrt(d0_ref[...] + d1_ref[...] + 1.0)
        dis_ref[...] = dis
        xn_ref[...] = xf_ref[...] * dis

    return pl.pallas_call(
        body,
        out_shape=[
            jax.ShapeDtypeStruct((NF, 128), jnp.float32),
            jax.ShapeDtypeStruct((NF, 128), jnp.float32),
        ],
    )(d0f, d1f, xf)


def _stage_b(a0f, a1f, xnf, disf, bd1, b1f, bd2):
    def body(a0_ref, a1_ref, xnf_ref, disf_ref, bd1_ref, b1f_ref, bd2_ref,
             hn2_ref):
        dis = disf_ref[...]
        g = (a0_ref[...] + a1_ref[...] + xnf_ref[...]) * dis
        h1 = jnp.dot(g, bd1_ref[...], preferred_element_type=jnp.float32)
        h1 = jnp.maximum(h1 + b1f_ref[...], 0.0)
        h2 = jnp.dot(h1, bd2_ref[...], preferred_element_type=jnp.float32)
        hn2_ref[...] = h2 * dis

    return pl.pallas_call(
        body,
        out_shape=jax.ShapeDtypeStruct((NF, 128), jnp.float32),
    )(a0f, a1f, xnf, disf, bd1, b1f, bd2)


def _stage_c(a0f, a1f, hn2f, disf, b2f):
    def body(a0_ref, a1_ref, hn2f_ref, disf_ref, b2f_ref, out_ref):
        out_ref[...] = ((a0_ref[...] + a1_ref[...] + hn2f_ref[...])
                        * disf_ref[...] + b2f_ref[...])

    return pl.pallas_call(
        body,
        out_shape=jax.ShapeDtypeStruct((NF, 128), jnp.float32),
    )(a0f, a1f, hn2f, disf, b2f)


_deg_call = _make_deg()
_agg8_call = _make_agg(8)


def kernel(x, edge_index, W1, b1, W2, b2):
    src = edge_index[0]
    dst = edge_index[1]
    pad = jnp.full((E_PAD - E,), N, jnp.int32)
    srcr = jnp.concatenate([src, pad]).reshape(EROWS, 128)
    dstr = jnp.concatenate([dst, pad]).reshape(EROWS, 128)
    xf = jnp.pad(x, ((0, N_PAD - N), (0, 5))).reshape(NF, 128)
    w1p = jnp.pad(W1, ((0, 5), (0, 0)))
    w2p = jnp.pad(W2, ((0, 0), (0, 1)))
    eye16 = jnp.eye(16, dtype=jnp.float32)
    bd1 = jnp.kron(eye16, w1p)
    bd2 = jnp.kron(eye16, w2p)
    b1f = jnp.tile(b1, 16).reshape(1, 256)
    b2f = jnp.tile(jnp.pad(b2, (0, 1)), 16).reshape(1, 128)
    ones = jnp.ones((128, 8), jnp.float32)
    z8 = jnp.zeros((R16, 8), jnp.float32)

    d0, d1 = _deg_call(dstr, ones, z8)
    disf, xnf = _stage_a(d0.reshape(NF, 128), d1.reshape(NF, 128), xf)
    a10, a11 = _agg8_call(xnf.reshape(N_PAD, 8), srcr, dstr, z8)
    hn2f = _stage_b(a10.reshape(NF, 128), a11.reshape(NF, 128), xnf, disf,
                    bd1, b1f, bd2)
    a20, a21 = _agg8_call(hn2f.reshape(N_PAD, 8), srcr, dstr, z8)
    outf = _stage_c(a20.reshape(NF, 128), a21.reshape(NF, 128), hn2f, disf,
                    b2f)
    return outf.reshape(N_PAD, 8)[:N, :7]
